# Initial kernel scaffold; baseline (speedup 1.0000x reference)
import jax
import jax.numpy as jnp
from jax.experimental import pallas as pl


def kernel(x, edge_index, W_gat, a_src, a_dst, gamma, beta, W1, b1, W2, b2, W3, b3):
    raise NotImplementedError("write your pallas kernel here")


# speedup vs baseline: 1.0489x; 1.0489x over previous
import functools

import jax
import jax.numpy as jnp
from jax.experimental import pallas as pl
from jax.experimental.pallas import tpu as pltpu

N = 10000
E = 320000
D = 128
H = 8
F = 8
HF = H * F
FC = 64
OMIC = 32
CLS = 2

_BK = 16000
_NBLK = (N * HF) // _BK


def _elu(v):
    return jnp.where(v > 0, v, jnp.exp(jnp.minimum(v, 0.0)) - 1.0)


def _head_body(flat_ref, W1_ref, b1_ref, W2_ref, b2_ref, W3_ref, b3_ref,
               g_ref, f_ref, p_ref, acc_ref):
    k = pl.program_id(0)

    @pl.when(k == 0)
    def _init():
        acc_ref[...] = jnp.zeros_like(acc_ref)

    acc_ref[...] += jnp.dot(flat_ref[...], W1_ref[...],
                            preferred_element_type=jnp.float32)

    @pl.when(k == _NBLK - 1)
    def _fin():
        g = _elu(acc_ref[...] + b1_ref[...])
        f = _elu(jnp.dot(g, W2_ref[...], preferred_element_type=jnp.float32)
                 + b2_ref[...])
        p = jnp.dot(f, W3_ref[...], preferred_element_type=jnp.float32) + b3_ref[...]
        g_ref[...] = g
        f_ref[...] = f
        p_ref[...] = p


@jax.jit
def _head(flat, W1, b1, W2, b2, W3, b3):
    return pl.pallas_call(
        _head_body,
        grid=(_NBLK,),
        in_specs=[
            pl.BlockSpec((1, _BK), lambda k: (0, k)),
            pl.BlockSpec((_BK, FC), lambda k: (k, 0)),
            pl.BlockSpec((1, FC), lambda k: (0, 0)),
            pl.BlockSpec((FC, OMIC), lambda k: (0, 0)),
            pl.BlockSpec((1, OMIC), lambda k: (0, 0)),
            pl.BlockSpec((OMIC, CLS), lambda k: (0, 0)),
            pl.BlockSpec((1, CLS), lambda k: (0, 0)),
        ],
        out_specs=[
            pl.BlockSpec((1, FC), lambda k: (0, 0)),
            pl.BlockSpec((1, OMIC), lambda k: (0, 0)),
            pl.BlockSpec((1, CLS), lambda k: (0, 0)),
        ],
        out_shape=[
            jax.ShapeDtypeStruct((1, FC), jnp.float32),
            jax.ShapeDtypeStruct((1, OMIC), jnp.float32),
            jax.ShapeDtypeStruct((1, CLS), jnp.float32),
        ],
        scratch_shapes=[pltpu.VMEM((1, FC), jnp.float32)],
    )(flat, W1, b1.reshape(1, FC), W2, b2.reshape(1, OMIC), W3,
      b3.reshape(1, CLS))


def _gat_conv_jax(x, edge_index, W_gat, a_src, a_dst, gamma, beta):
    h = (x @ W_gat).reshape(N, H, F)
    src = edge_index[0]
    dst = edge_index[1]
    alpha_src = jnp.sum(h * a_src[None, :, :], axis=-1)
    alpha_dst = jnp.sum(h * a_dst[None, :, :], axis=-1)
    e = jax.nn.leaky_relu(alpha_src[src] + alpha_dst[dst], negative_slope=0.2)
    C = jnp.max(alpha_src, axis=0) + jnp.max(alpha_dst, axis=0)
    C = jnp.maximum(C, 0.0)
    ex = jnp.exp(e - C[None, :])
    denom = jax.ops.segment_sum(ex, dst, num_segments=N)
    numer = jax.ops.segment_sum(ex[:, :, None] * h[src], dst, num_segments=N)
    out = numer / (denom[:, :, None] + 1e-16)
    out = _elu(out).reshape(N, HF)
    mu = jnp.mean(out, axis=-1, keepdims=True)
    var = jnp.var(out, axis=-1, keepdims=True)
    out = (out - mu) / jnp.sqrt(var + 1e-5) * gamma + beta
    return out


def kernel(x, edge_index, W_gat, a_src, a_dst, gamma, beta,
           W1, b1, W2, b2, W3, b3):
    out = _gat_conv_jax(x, edge_index, W_gat, a_src, a_dst, gamma, beta)
    flat = out.reshape(1, N * HF)
    g, f, p = _head(flat, W1, b1, W2, b2, W3, b3)
    pred = p
    Y_hat = jnp.argmax(pred, axis=1)
    Y_prob = jax.nn.softmax(pred, axis=1)
    return pred, Y_prob, Y_hat, g.reshape(FC), f.reshape(OMIC)


# speedup vs baseline: 21.0740x; 20.0910x over previous
import functools

import jax
import jax.numpy as jnp
from jax import lax
from jax.experimental import pallas as pl
from jax.experimental.pallas import tpu as pltpu
from jax.experimental.pallas import tpu_sc as plsc

N = 10000
E = 320000
D = 128
H = 8
F = 8
HF = H * F
FC = 64
OMIC = 32
CLS = 2

_TW = 128
_DW = 16
_NC = 2
_NS = 16
_NW = _NC * _NS
_ETP = E // _NS
_CH = 80
_NCH = _ETP // _CH
_NP = 2
_AR = 2560
_ZR = _AR // _NS


def _elu(v):
    return jnp.where(v > 0, v, jnp.exp(jnp.minimum(v, 0.0)) - 1.0)


_BN = 1000
_NBN = N // _BN


def _tabs_body(x_ref, wg_ref, as_ref, ad_ref, st_ref, dt_ref, c16_ref,
               ms_ref, md_ref):
    k = pl.program_id(0)
    h = jnp.dot(x_ref[...], wg_ref[...], preferred_element_type=jnp.float32)
    asb = jnp.dot(h, as_ref[...], preferred_element_type=jnp.float32)
    adb = jnp.dot(h, ad_ref[...], preferred_element_type=jnp.float32)
    z56 = jnp.zeros((_BN, _TW - HF - H), jnp.float32)
    st_ref[...] = jnp.concatenate([h, asb, z56], axis=1)
    z120 = jnp.zeros((_BN, _TW - H), jnp.float32)
    dt_ref[...] = jnp.concatenate([adb, z120], axis=1)
    bs = jnp.max(asb, axis=0, keepdims=True)
    bd = jnp.max(adb, axis=0, keepdims=True)

    @pl.when(k == 0)
    def _first():
        ms_ref[...] = bs
        md_ref[...] = bd

    @pl.when(k > 0)
    def _rest():
        ms_ref[...] = jnp.maximum(ms_ref[...], bs)
        md_ref[...] = jnp.maximum(md_ref[...], bd)

    @pl.when(k == _NBN - 1)
    def _fin():
        cv = jnp.maximum(ms_ref[...] + md_ref[...], 0.0)
        c16_ref[...] = jnp.concatenate(
            [cv, jnp.full((1, 8), 1e30, jnp.float32)], axis=1)


@jax.jit
def _tabs(x, wg, As, Ad):
    return pl.pallas_call(
        _tabs_body,
        grid=(_NBN,),
        in_specs=[
            pl.BlockSpec((_BN, D), lambda k: (k, 0)),
            pl.BlockSpec((D, HF), lambda k: (0, 0)),
            pl.BlockSpec((HF, H), lambda k: (0, 0)),
            pl.BlockSpec((HF, H), lambda k: (0, 0)),
        ],
        out_specs=[
            pl.BlockSpec((_BN, _TW), lambda k: (k, 0)),
            pl.BlockSpec((_BN, _TW), lambda k: (k, 0)),
            pl.BlockSpec((1, _DW), lambda k: (0, 0)),
        ],
        out_shape=[
            jax.ShapeDtypeStruct((N, _TW), jnp.float32),
            jax.ShapeDtypeStruct((N, _TW), jnp.float32),
            jax.ShapeDtypeStruct((1, _DW), jnp.float32),
        ],
        scratch_shapes=[pltpu.VMEM((1, H), jnp.float32),
                        pltpu.VMEM((1, H), jnp.float32)],
    )(x, wg, As, Ad)


def _edge_body(src_hbm, dst_hbm, stab_hbm, dtab_hbm, c16_hbm, zeros_hbm,
               out_hbm, sidx, didx, lsidx, lidx, ldidx, gbuf, dbuf, cbuf,
               acc, sem1, sem2):
    c = lax.axis_index("c")
    s = lax.axis_index("s")

    pltpu.sync_copy(c16_hbm, cbuf)
    cv = cbuf[...]
    io = lax.iota(jnp.int32, 16)
    hi = jnp.where(io >= 8, 1, 0)

    for p in range(_NP):
        rng = 2 * p + c
        nbase = pl.multiple_of(rng * _AR, 8)
        pltpu.sync_copy(zeros_hbm.at[pl.ds(s * _ZR, _ZR)],
                        acc.at[pl.ds(s * _ZR, _ZR)])
        plsc.subcore_barrier()

        def chunk_body(chunk, carry0):
            base = pl.multiple_of(s * _ETP, 8) + chunk * _CH
            pltpu.sync_copy(src_hbm.at[pl.ds(base, _CH)], sidx)
            pltpu.sync_copy(dst_hbm.at[pl.ds(base, _CH)], didx)

            def mask_body(j, carry1):
                o16 = pl.multiple_of(j * 16, 16)
                dvv = didx[pl.ds(o16, 16)]
                svv = sidx[pl.ds(o16, 16)]
                lv = dvv - nbase
                ok = jnp.logical_and(lv >= 0, lv < _AR)
                lidx[pl.ds(o16, 16)] = jnp.where(ok, lv, -1)
                lsidx[pl.ds(o16, 16)] = jnp.where(ok, svv, -1)
                ldidx[pl.ds(o16, 16)] = jnp.where(ok, dvv, -1)
                return carry1

            lax.fori_loop(0, _CH // 16, mask_body, 0)
            cp1 = pltpu.async_copy(
                stab_hbm.at[plsc.Indices(lsidx, ignored_value=-1)],
                gbuf, sem1)
            cp2 = pltpu.async_copy(
                dtab_hbm.at[plsc.Indices(ldidx, ignored_value=-1)],
                dbuf, sem2)
            cp1.wait()
            cp2.wait()

            def body(i, carry):
                dv = dbuf[i, pl.ds(0, 16)]
                av = gbuf[i, pl.ds(HF, 16)]
                sv = av + dv
                lkv = jnp.maximum(sv, 0.2 * sv)
                w = jnp.exp(lkv - cv)
                gbuf[i, pl.ds(HF, 16)] = w
                for u in range(4):
                    hv = gbuf[i, pl.ds(u * 16, 16)]
                    pat = hi + 2 * u
                    wx = jnp.take_along_axis(w, pat, axis=0,
                                             mode="promise_in_bounds")
                    gbuf[i, pl.ds(u * 16, 16)] = hv * wx
                return carry

            lax.fori_loop(0, _CH, body, 0)
            pltpu.sync_copy(gbuf,
                            acc.at[plsc.Indices(lidx, ignored_value=-1)],
                            add=True)
            return carry0

        lax.fori_loop(0, _NCH, chunk_body, 0)

        plsc.subcore_barrier()
        pltpu.sync_copy(acc.at[pl.ds(s * _ZR, _ZR)],
                        out_hbm.at[rng, pl.ds(s * _ZR, _ZR)])
        plsc.subcore_barrier()


@jax.jit
def _edge(src, dst, stab, dtab, c16, zeros2560):
    mesh = plsc.VectorSubcoreMesh(core_axis_name="c", subcore_axis_name="s")
    fn = functools.partial(
        pl.kernel,
        out_type=jax.ShapeDtypeStruct((_NP * _NC, _AR, _TW), jnp.float32),
        mesh=mesh,
        scratch_types=[
            pltpu.VMEM((_CH,), jnp.int32),
            pltpu.VMEM((_CH,), jnp.int32),
            pltpu.VMEM((_CH,), jnp.int32),
            pltpu.VMEM((_CH,), jnp.int32),
            pltpu.VMEM((_CH,), jnp.int32),
            pltpu.VMEM((_CH, _TW), jnp.float32),
            pltpu.VMEM((_CH, _TW), jnp.float32),
            pltpu.VMEM((_DW,), jnp.float32),
            pltpu.VMEM_SHARED((_AR, _TW), jnp.float32),
            pltpu.SemaphoreType.DMA,
            pltpu.SemaphoreType.DMA,
        ],
    )(_edge_body)
    return fn(src, dst, stab, dtab, c16, zeros2560)


def _post_body(a_ref, r_ref, g_ref, b_ref, o_ref):
    a = a_ref[...]
    numer = a[:, :HF]
    den = a[:, HF:HF + H]
    dexp = jnp.dot(den, r_ref[...], preferred_element_type=jnp.float32)
    o = numer / (dexp + 1e-16)
    o = _elu(o)
    mu = jnp.mean(o, axis=1, keepdims=True)
    var = jnp.mean((o - mu) ** 2, axis=1, keepdims=True)
    o_ref[...] = ((o - mu) * lax.rsqrt(var + 1e-5)) * g_ref[...] + b_ref[...]


@jax.jit
def _post(a, R, gamma, beta):
    return pl.pallas_call(
        _post_body,
        grid=(_NBN,),
        in_specs=[
            pl.BlockSpec((_BN, _TW), lambda k: (k, 0)),
            pl.BlockSpec((H, HF), lambda k: (0, 0)),
            pl.BlockSpec((1, HF), lambda k: (0, 0)),
            pl.BlockSpec((1, HF), lambda k: (0, 0)),
        ],
        out_specs=pl.BlockSpec((_BN, HF), lambda k: (k, 0)),
        out_shape=jax.ShapeDtypeStruct((N, HF), jnp.float32),
    )(a, R, gamma, beta)


_BK = 16000
_NBLK = (N * HF) // _BK


def _head_body(flat_ref, W1_ref, b1_ref, W2_ref, b2_ref, W3_ref, b3_ref,
               g_ref, f_ref, p_ref, acc_ref):
    k = pl.program_id(0)

    @pl.when(k == 0)
    def _init():
        acc_ref[...] = jnp.zeros_like(acc_ref)

    acc_ref[...] += jnp.dot(flat_ref[...], W1_ref[...],
                            preferred_element_type=jnp.float32)

    @pl.when(k == _NBLK - 1)
    def _fin():
        g = _elu(acc_ref[...] + b1_ref[...])
        f = _elu(jnp.dot(g, W2_ref[...], preferred_element_type=jnp.float32)
                 + b2_ref[...])
        p = jnp.dot(f, W3_ref[...], preferred_element_type=jnp.float32) + b3_ref[...]
        g_ref[...] = g
        f_ref[...] = f
        p_ref[...] = p


@jax.jit
def _head(flat, W1, b1, W2, b2, W3, b3):
    return pl.pallas_call(
        _head_body,
        grid=(_NBLK,),
        in_specs=[
            pl.BlockSpec((1, _BK), lambda k: (0, k)),
            pl.BlockSpec((_BK, FC), lambda k: (k, 0)),
            pl.BlockSpec((1, FC), lambda k: (0, 0)),
            pl.BlockSpec((FC, OMIC), lambda k: (0, 0)),
            pl.BlockSpec((1, OMIC), lambda k: (0, 0)),
            pl.BlockSpec((OMIC, CLS), lambda k: (0, 0)),
            pl.BlockSpec((1, CLS), lambda k: (0, 0)),
        ],
        out_specs=[
            pl.BlockSpec((1, FC), lambda k: (0, 0)),
            pl.BlockSpec((1, OMIC), lambda k: (0, 0)),
            pl.BlockSpec((1, CLS), lambda k: (0, 0)),
        ],
        out_shape=[
            jax.ShapeDtypeStruct((1, FC), jnp.float32),
            jax.ShapeDtypeStruct((1, OMIC), jnp.float32),
            jax.ShapeDtypeStruct((1, CLS), jnp.float32),
        ],
        scratch_shapes=[pltpu.VMEM((1, FC), jnp.float32)],
    )(flat, W1, b1.reshape(1, FC), W2, b2.reshape(1, OMIC), W3,
      b3.reshape(1, CLS))


def kernel(x, edge_index, W_gat, a_src, a_dst, gamma, beta,
           W1, b1, W2, b2, W3, b3):
    src = edge_index[0]
    dst = edge_index[1]
    rows = jnp.arange(HF)
    cols = rows // F
    As = jnp.zeros((HF, H), jnp.float32).at[rows, cols].set(a_src.reshape(HF))
    Ad = jnp.zeros((HF, H), jnp.float32).at[rows, cols].set(a_dst.reshape(HF))
    stab, dtab, c16 = _tabs(x, W_gat, As, Ad)
    zeros2560 = jnp.zeros((_AR, _TW), jnp.float32)
    acc4 = _edge(src, dst, stab, dtab, c16.reshape(_DW), zeros2560)
    acc = acc4.reshape(_NP * _NC * _AR, _TW)[:N]
    R = jnp.kron(jnp.eye(H, dtype=jnp.float32), jnp.ones((1, F), jnp.float32))
    ln = _post(acc, R, gamma.reshape(1, HF), beta.reshape(1, HF))
    flat = ln.reshape(1, N * HF)
    g, f, p = _head(flat, W1, b1, W2, b2, W3, b3)
    pred = p
    Y_hat = jnp.argmax(pred, axis=1)
    Y_prob = jax.nn.softmax(pred, axis=1)
    return pred, Y_prob, Y_hat, g.reshape(FC), f.reshape(OMIC)


# speedup vs baseline: 30.4240x; 1.4437x over previous
import functools

import jax
import jax.numpy as jnp
from jax import lax
from jax.experimental import pallas as pl
from jax.experimental.pallas import tpu as pltpu
from jax.experimental.pallas import tpu_sc as plsc

N = 10000
E = 320000
D = 128
H = 8
F = 8
HF = H * F
FC = 64
OMIC = 32
CLS = 2

_TW = 128
_DW = 16
_NC = 2
_NS = 16
_NW = _NC * _NS
_ETP = E // _NS
_SB = 80
_NSB = 5
_CH = _SB * _NSB
_NCH = _ETP // _CH
_NP = 2
_AR = 2560
_ZR = _AR // _NS


def _elu(v):
    return jnp.where(v > 0, v, jnp.exp(jnp.minimum(v, 0.0)) - 1.0)


_BN = 1000
_NBN = N // _BN


def _tabs_body(x_ref, wg_ref, as_ref, ad_ref, st_ref, dt_ref, c16_ref,
               ms_ref, md_ref):
    k = pl.program_id(0)
    h = jnp.dot(x_ref[...], wg_ref[...], preferred_element_type=jnp.float32)
    asb = jnp.dot(h, as_ref[...], preferred_element_type=jnp.float32)
    adb = jnp.dot(h, ad_ref[...], preferred_element_type=jnp.float32)
    z56 = jnp.zeros((_BN, _TW - HF - H), jnp.float32)
    st_ref[...] = jnp.concatenate([h, asb, z56], axis=1)
    z120 = jnp.zeros((_BN, _TW - H), jnp.float32)
    dt_ref[...] = jnp.concatenate([adb, z120], axis=1)
    bs = jnp.max(asb, axis=0, keepdims=True)
    bd = jnp.max(adb, axis=0, keepdims=True)

    @pl.when(k == 0)
    def _first():
        ms_ref[...] = bs
        md_ref[...] = bd

    @pl.when(k > 0)
    def _rest():
        ms_ref[...] = jnp.maximum(ms_ref[...], bs)
        md_ref[...] = jnp.maximum(md_ref[...], bd)

    @pl.when(k == _NBN - 1)
    def _fin():
        cv = jnp.maximum(ms_ref[...] + md_ref[...], 0.0)
        c16_ref[...] = jnp.concatenate(
            [cv, jnp.full((1, 8), 1e30, jnp.float32)], axis=1)


@jax.jit
def _tabs(x, wg, As, Ad):
    return pl.pallas_call(
        _tabs_body,
        grid=(_NBN,),
        in_specs=[
            pl.BlockSpec((_BN, D), lambda k: (k, 0)),
            pl.BlockSpec((D, HF), lambda k: (0, 0)),
            pl.BlockSpec((HF, H), lambda k: (0, 0)),
            pl.BlockSpec((HF, H), lambda k: (0, 0)),
        ],
        out_specs=[
            pl.BlockSpec((_BN, _TW), lambda k: (k, 0)),
            pl.BlockSpec((_BN, _TW), lambda k: (k, 0)),
            pl.BlockSpec((1, _DW), lambda k: (0, 0)),
        ],
        out_shape=[
            jax.ShapeDtypeStruct((N, _TW), jnp.float32),
            jax.ShapeDtypeStruct((N, _TW), jnp.float32),
            jax.ShapeDtypeStruct((1, _DW), jnp.float32),
        ],
        scratch_shapes=[pltpu.VMEM((1, H), jnp.float32),
                        pltpu.VMEM((1, H), jnp.float32)],
    )(x, wg, As, Ad)


def _edge_body(src_hbm, dst_hbm, stab_hbm, dtab_hbm, c16_hbm, zeros_hbm,
               out_hbm, sidx, didx, lsidx, lidx, ldidx, gbuf, dbuf, cbuf,
               acc, sem1, sem2, sem3):
    c = lax.axis_index("c")
    s = lax.axis_index("s")

    pltpu.sync_copy(c16_hbm, cbuf)
    cv = cbuf[...]
    io = lax.iota(jnp.int32, 16)
    hi = jnp.where(io >= 8, 1, 0)

    for p in range(_NP):
        rng = 2 * p + c
        nbase = pl.multiple_of(rng * _AR, 8)
        pltpu.sync_copy(zeros_hbm.at[pl.ds(s * _ZR, _ZR)],
                        acc.at[pl.ds(s * _ZR, _ZR)])
        plsc.subcore_barrier()

        def chunk_body(chunk, carry0):
            base = pl.multiple_of(s * _ETP, 8) + chunk * _CH
            pltpu.sync_copy(src_hbm.at[pl.ds(base, _CH)], sidx)
            pltpu.sync_copy(dst_hbm.at[pl.ds(base, _CH)], didx)

            def mask_body(j, carry1):
                o16 = pl.multiple_of(j * 16, 16)
                dvv = didx[pl.ds(o16, 16)]
                svv = sidx[pl.ds(o16, 16)]
                lv = dvv - nbase
                ok = jnp.logical_and(lv >= 0, lv < _AR)
                q = j // (_SB // 16)
                r16 = pl.multiple_of((j % (_SB // 16)) * 16, 16)
                lidx[q, pl.ds(r16, 16)] = jnp.where(ok, lv, -1)
                lsidx[q, pl.ds(r16, 16)] = jnp.where(ok, svv, -1)
                ldidx[q, pl.ds(r16, 16)] = jnp.where(ok, dvv, -1)
                return carry1

            lax.fori_loop(0, _CH // 16, mask_body, 0)
            cps = []
            for q in range(_NSB):
                cps.append(pltpu.async_copy(
                    stab_hbm.at[plsc.Indices(lsidx.at[q], ignored_value=-1)],
                    gbuf.at[pl.ds(q * _SB, _SB)], sem1))
                cps.append(pltpu.async_copy(
                    dtab_hbm.at[plsc.Indices(ldidx.at[q], ignored_value=-1)],
                    dbuf.at[pl.ds(q * _SB, _SB)], sem2))
            for cp in cps:
                cp.wait()

            def body(i, carry):
                dv = dbuf[i, pl.ds(0, 16)]
                av = gbuf[i, pl.ds(HF, 16)]
                sv = av + dv
                lkv = jnp.maximum(sv, 0.2 * sv)
                w = jnp.exp(lkv - cv)
                gbuf[i, pl.ds(HF, 16)] = w
                for u in range(4):
                    hv = gbuf[i, pl.ds(u * 16, 16)]
                    pat = hi + 2 * u
                    wx = jnp.take_along_axis(w, pat, axis=0,
                                             mode="promise_in_bounds")
                    gbuf[i, pl.ds(u * 16, 16)] = hv * wx
                return carry

            lax.fori_loop(0, _CH, body, 0)
            scs = []
            for q in range(_NSB):
                scs.append(pltpu.async_copy(
                    gbuf.at[pl.ds(q * _SB, _SB)],
                    acc.at[plsc.Indices(lidx.at[q], ignored_value=-1)],
                    sem3, add=True))
            for cp in scs:
                cp.wait()
            return carry0

        lax.fori_loop(0, _NCH, chunk_body, 0)

        plsc.subcore_barrier()
        pltpu.sync_copy(acc.at[pl.ds(s * _ZR, _ZR)],
                        out_hbm.at[rng, pl.ds(s * _ZR, _ZR)])
        plsc.subcore_barrier()


@jax.jit
def _edge(src, dst, stab, dtab, c16, zeros2560):
    mesh = plsc.VectorSubcoreMesh(core_axis_name="c", subcore_axis_name="s")
    fn = functools.partial(
        pl.kernel,
        out_type=jax.ShapeDtypeStruct((_NP * _NC, _AR, _TW), jnp.float32),
        mesh=mesh,
        scratch_types=[
            pltpu.VMEM((_CH,), jnp.int32),
            pltpu.VMEM((_CH,), jnp.int32),
            pltpu.VMEM((_NSB, _SB), jnp.int32),
            pltpu.VMEM((_NSB, _SB), jnp.int32),
            pltpu.VMEM((_NSB, _SB), jnp.int32),
            pltpu.VMEM((_CH, _TW), jnp.float32),
            pltpu.VMEM((_CH, _TW), jnp.float32),
            pltpu.VMEM((_DW,), jnp.float32),
            pltpu.VMEM_SHARED((_AR, _TW), jnp.float32),
            pltpu.SemaphoreType.DMA,
            pltpu.SemaphoreType.DMA,
            pltpu.SemaphoreType.DMA,
        ],
    )(_edge_body)
    return fn(src, dst, stab, dtab, c16, zeros2560)


def _post_body(a_ref, r_ref, g_ref, b_ref, o_ref):
    a = a_ref[...]
    numer = a[:, :HF]
    den = a[:, HF:HF + H]
    dexp = jnp.dot(den, r_ref[...], preferred_element_type=jnp.float32)
    o = numer / (dexp + 1e-16)
    o = _elu(o)
    mu = jnp.mean(o, axis=1, keepdims=True)
    var = jnp.mean((o - mu) ** 2, axis=1, keepdims=True)
    o_ref[...] = ((o - mu) * lax.rsqrt(var + 1e-5)) * g_ref[...] + b_ref[...]


@jax.jit
def _post(a, R, gamma, beta):
    return pl.pallas_call(
        _post_body,
        grid=(_NBN,),
        in_specs=[
            pl.BlockSpec((_BN, _TW), lambda k: (k, 0)),
            pl.BlockSpec((H, HF), lambda k: (0, 0)),
            pl.BlockSpec((1, HF), lambda k: (0, 0)),
            pl.BlockSpec((1, HF), lambda k: (0, 0)),
        ],
        out_specs=pl.BlockSpec((_BN, HF), lambda k: (k, 0)),
        out_shape=jax.ShapeDtypeStruct((N, HF), jnp.float32),
    )(a, R, gamma, beta)


_BK = 16000
_NBLK = (N * HF) // _BK


def _head_body(flat_ref, W1_ref, b1_ref, W2_ref, b2_ref, W3_ref, b3_ref,
               g_ref, f_ref, p_ref, acc_ref):
    k = pl.program_id(0)

    @pl.when(k == 0)
    def _init():
        acc_ref[...] = jnp.zeros_like(acc_ref)

    acc_ref[...] += jnp.dot(flat_ref[...], W1_ref[...],
                            preferred_element_type=jnp.float32)

    @pl.when(k == _NBLK - 1)
    def _fin():
        g = _elu(acc_ref[...] + b1_ref[...])
        f = _elu(jnp.dot(g, W2_ref[...], preferred_element_type=jnp.float32)
                 + b2_ref[...])
        p = jnp.dot(f, W3_ref[...], preferred_element_type=jnp.float32) + b3_ref[...]
        g_ref[...] = g
        f_ref[...] = f
        p_ref[...] = p


@jax.jit
def _head(flat, W1, b1, W2, b2, W3, b3):
    return pl.pallas_call(
        _head_body,
        grid=(_NBLK,),
        in_specs=[
            pl.BlockSpec((1, _BK), lambda k: (0, k)),
            pl.BlockSpec((_BK, FC), lambda k: (k, 0)),
            pl.BlockSpec((1, FC), lambda k: (0, 0)),
            pl.BlockSpec((FC, OMIC), lambda k: (0, 0)),
            pl.BlockSpec((1, OMIC), lambda k: (0, 0)),
            pl.BlockSpec((OMIC, CLS), lambda k: (0, 0)),
            pl.BlockSpec((1, CLS), lambda k: (0, 0)),
        ],
        out_specs=[
            pl.BlockSpec((1, FC), lambda k: (0, 0)),
            pl.BlockSpec((1, OMIC), lambda k: (0, 0)),
            pl.BlockSpec((1, CLS), lambda k: (0, 0)),
        ],
        out_shape=[
            jax.ShapeDtypeStruct((1, FC), jnp.float32),
            jax.ShapeDtypeStruct((1, OMIC), jnp.float32),
            jax.ShapeDtypeStruct((1, CLS), jnp.float32),
        ],
        scratch_shapes=[pltpu.VMEM((1, FC), jnp.float32)],
    )(flat, W1, b1.reshape(1, FC), W2, b2.reshape(1, OMIC), W3,
      b3.reshape(1, CLS))


def kernel(x, edge_index, W_gat, a_src, a_dst, gamma, beta,
           W1, b1, W2, b2, W3, b3):
    src = edge_index[0]
    dst = edge_index[1]
    rows = jnp.arange(HF)
    cols = rows // F
    As = jnp.zeros((HF, H), jnp.float32).at[rows, cols].set(a_src.reshape(HF))
    Ad = jnp.zeros((HF, H), jnp.float32).at[rows, cols].set(a_dst.reshape(HF))
    stab, dtab, c16 = _tabs(x, W_gat, As, Ad)
    zeros2560 = jnp.zeros((_AR, _TW), jnp.float32)
    acc4 = _edge(src, dst, stab, dtab, c16.reshape(_DW), zeros2560)
    acc = acc4.reshape(_NP * _NC * _AR, _TW)[:N]
    R = jnp.kron(jnp.eye(H, dtype=jnp.float32), jnp.ones((1, F), jnp.float32))
    ln = _post(acc, R, gamma.reshape(1, HF), beta.reshape(1, HF))
    flat = ln.reshape(1, N * HF)
    g, f, p = _head(flat, W1, b1, W2, b2, W3, b3)
    pred = p
    Y_hat = jnp.argmax(pred, axis=1)
    Y_prob = jax.nn.softmax(pred, axis=1)
    return pred, Y_prob, Y_hat, g.reshape(FC), f.reshape(OMIC)


# speedup vs baseline: 45.4996x; 1.4955x over previous
import functools

import jax
import jax.numpy as jnp
from jax import lax
from jax.experimental import pallas as pl
from jax.experimental.pallas import tpu as pltpu
from jax.experimental.pallas import tpu_sc as plsc

N = 10000
E = 320000
D = 128
H = 8
F = 8
HF = H * F
FC = 64
OMIC = 32
CLS = 2

_TW = 128
_DW = 16
_NC = 2
_NS = 16
_NW = _NC * _NS
_ETP = E // _NS
_SB = 80
_NSB = 5
_CH = _SB * _NSB
_NCH = _ETP // _CH
_NP = 2
_AR = 2560
_ZR = _AR // _NS


def _elu(v):
    return jnp.where(v > 0, v, jnp.exp(jnp.minimum(v, 0.0)) - 1.0)


_BN = 1000
_NBN = N // _BN


def _tabs_body(x_ref, wg_ref, as_ref, ad_ref, st_ref, dt_ref, c16_ref,
               ms_ref, md_ref):
    k = pl.program_id(0)
    h = jnp.dot(x_ref[...], wg_ref[...], preferred_element_type=jnp.float32)
    asb = jnp.dot(h, as_ref[...], preferred_element_type=jnp.float32)
    adb = jnp.dot(h, ad_ref[...], preferred_element_type=jnp.float32)
    z56 = jnp.zeros((_BN, _TW - HF - H), jnp.float32)
    st_ref[...] = jnp.concatenate([h, asb, z56], axis=1)
    z120 = jnp.zeros((_BN, _TW - H), jnp.float32)
    dt_ref[...] = jnp.concatenate([adb, z120], axis=1)
    bs = jnp.max(asb, axis=0, keepdims=True)
    bd = jnp.max(adb, axis=0, keepdims=True)

    @pl.when(k == 0)
    def _first():
        ms_ref[...] = bs
        md_ref[...] = bd

    @pl.when(k > 0)
    def _rest():
        ms_ref[...] = jnp.maximum(ms_ref[...], bs)
        md_ref[...] = jnp.maximum(md_ref[...], bd)

    @pl.when(k == _NBN - 1)
    def _fin():
        cv = jnp.maximum(ms_ref[...] + md_ref[...], 0.0)
        c16_ref[...] = jnp.concatenate(
            [cv, jnp.full((1, 8), 1e30, jnp.float32)], axis=1)


@jax.jit
def _tabs(x, wg, As, Ad):
    return pl.pallas_call(
        _tabs_body,
        grid=(_NBN,),
        in_specs=[
            pl.BlockSpec((_BN, D), lambda k: (k, 0)),
            pl.BlockSpec((D, HF), lambda k: (0, 0)),
            pl.BlockSpec((HF, H), lambda k: (0, 0)),
            pl.BlockSpec((HF, H), lambda k: (0, 0)),
        ],
        out_specs=[
            pl.BlockSpec((_BN, _TW), lambda k: (k, 0)),
            pl.BlockSpec((_BN, _TW), lambda k: (k, 0)),
            pl.BlockSpec((1, _DW), lambda k: (0, 0)),
        ],
        out_shape=[
            jax.ShapeDtypeStruct((N, _TW), jnp.float32),
            jax.ShapeDtypeStruct((N, _TW), jnp.float32),
            jax.ShapeDtypeStruct((1, _DW), jnp.float32),
        ],
        scratch_shapes=[pltpu.VMEM((1, H), jnp.float32),
                        pltpu.VMEM((1, H), jnp.float32)],
    )(x, wg, As, Ad)


def _edge_body(src_hbm, dst_hbm, stab_hbm, dtab_hbm, c16_hbm, zeros_hbm,
               out_hbm, sidx, didx, lsidx, lidx, ldidx, gbuf, dbuf, cbuf,
               acc, sem1, sem2, sem3):
    c = lax.axis_index("c")
    s = lax.axis_index("s")

    pltpu.sync_copy(c16_hbm, cbuf)
    cv = cbuf[...]
    io = lax.iota(jnp.int32, 16)
    hi = jnp.where(io >= 8, 1, 0)

    for p in range(_NP):
        rng = 2 * p + c
        nbase = pl.multiple_of(rng * _AR, 8)
        pltpu.sync_copy(zeros_hbm.at[pl.ds(s * _ZR, _ZR)],
                        acc.at[pl.ds(s * _ZR, _ZR)])
        plsc.subcore_barrier()

        def chunk_body(chunk, carry0):
            base = pl.multiple_of(s * _ETP, 8) + chunk * _CH
            pltpu.sync_copy(src_hbm.at[pl.ds(base, _CH)], sidx)
            pltpu.sync_copy(dst_hbm.at[pl.ds(base, _CH)], didx)

            @plsc.parallel_loop(0, _CH // 16, unroll=5)
            def mask_body(j):
                o16 = pl.multiple_of(j * 16, 16)
                dvv = didx[pl.ds(o16, 16)]
                svv = sidx[pl.ds(o16, 16)]
                lv = dvv - nbase
                ok = jnp.logical_and(lv >= 0, lv < _AR)
                q = j // (_SB // 16)
                r16 = pl.multiple_of((j % (_SB // 16)) * 16, 16)
                lidx[q, pl.ds(r16, 16)] = jnp.where(ok, lv, -1)
                lsidx[q, pl.ds(r16, 16)] = jnp.where(ok, svv, -1)
                ldidx[q, pl.ds(r16, 16)] = jnp.where(ok, dvv, -1)
            cps = []
            for q in range(_NSB):
                cps.append(pltpu.async_copy(
                    stab_hbm.at[plsc.Indices(lsidx.at[q], ignored_value=-1)],
                    gbuf.at[pl.ds(q * _SB, _SB)], sem1))
                cps.append(pltpu.async_copy(
                    dtab_hbm.at[plsc.Indices(ldidx.at[q], ignored_value=-1)],
                    dbuf.at[pl.ds(q * _SB, _SB)], sem2))
            for cp in cps:
                cp.wait()

            @plsc.parallel_loop(0, _CH, unroll=4)
            def body(i):
                dv = dbuf[i, pl.ds(0, 16)]
                av = gbuf[i, pl.ds(HF, 16)]
                sv = av + dv
                lkv = jnp.maximum(sv, 0.2 * sv)
                w = jnp.exp(lkv - cv)
                gbuf[i, pl.ds(HF, 16)] = w
                for u in range(4):
                    hv = gbuf[i, pl.ds(u * 16, 16)]
                    pat = hi + 2 * u
                    wx = jnp.take_along_axis(w, pat, axis=0,
                                             mode="promise_in_bounds")
                    gbuf[i, pl.ds(u * 16, 16)] = hv * wx
            scs = []
            for q in range(_NSB):
                scs.append(pltpu.async_copy(
                    gbuf.at[pl.ds(q * _SB, _SB)],
                    acc.at[plsc.Indices(lidx.at[q], ignored_value=-1)],
                    sem3, add=True))
            for cp in scs:
                cp.wait()
            return carry0

        lax.fori_loop(0, _NCH, chunk_body, 0)

        plsc.subcore_barrier()
        pltpu.sync_copy(acc.at[pl.ds(s * _ZR, _ZR)],
                        out_hbm.at[rng, pl.ds(s * _ZR, _ZR)])
        plsc.subcore_barrier()


@jax.jit
def _edge(src, dst, stab, dtab, c16, zeros2560):
    mesh = plsc.VectorSubcoreMesh(core_axis_name="c", subcore_axis_name="s")
    fn = functools.partial(
        pl.kernel,
        out_type=jax.ShapeDtypeStruct((_NP * _NC, _AR, _TW), jnp.float32),
        mesh=mesh,
        scratch_types=[
            pltpu.VMEM((_CH,), jnp.int32),
            pltpu.VMEM((_CH,), jnp.int32),
            pltpu.VMEM((_NSB, _SB), jnp.int32),
            pltpu.VMEM((_NSB, _SB), jnp.int32),
            pltpu.VMEM((_NSB, _SB), jnp.int32),
            pltpu.VMEM((_CH, _TW), jnp.float32),
            pltpu.VMEM((_CH, _TW), jnp.float32),
            pltpu.VMEM((_DW,), jnp.float32),
            pltpu.VMEM_SHARED((_AR, _TW), jnp.float32),
            pltpu.SemaphoreType.DMA,
            pltpu.SemaphoreType.DMA,
            pltpu.SemaphoreType.DMA,
        ],
    )(_edge_body)
    return fn(src, dst, stab, dtab, c16, zeros2560)


def _post_body(a_ref, r_ref, g_ref, b_ref, o_ref):
    a = a_ref[...]
    numer = a[:, :HF]
    den = a[:, HF:HF + H]
    dexp = jnp.dot(den, r_ref[...], preferred_element_type=jnp.float32)
    o = numer / (dexp + 1e-16)
    o = _elu(o)
    mu = jnp.mean(o, axis=1, keepdims=True)
    var = jnp.mean((o - mu) ** 2, axis=1, keepdims=True)
    o_ref[...] = ((o - mu) * lax.rsqrt(var + 1e-5)) * g_ref[...] + b_ref[...]


@jax.jit
def _post(a, R, gamma, beta):
    return pl.pallas_call(
        _post_body,
        grid=(_NBN,),
        in_specs=[
            pl.BlockSpec((_BN, _TW), lambda k: (k, 0)),
            pl.BlockSpec((H, HF), lambda k: (0, 0)),
            pl.BlockSpec((1, HF), lambda k: (0, 0)),
            pl.BlockSpec((1, HF), lambda k: (0, 0)),
        ],
        out_specs=pl.BlockSpec((_BN, HF), lambda k: (k, 0)),
        out_shape=jax.ShapeDtypeStruct((N, HF), jnp.float32),
    )(a, R, gamma, beta)


_BK = 16000
_NBLK = (N * HF) // _BK


def _head_body(flat_ref, W1_ref, b1_ref, W2_ref, b2_ref, W3_ref, b3_ref,
               g_ref, f_ref, p_ref, acc_ref):
    k = pl.program_id(0)

    @pl.when(k == 0)
    def _init():
        acc_ref[...] = jnp.zeros_like(acc_ref)

    acc_ref[...] += jnp.dot(flat_ref[...], W1_ref[...],
                            preferred_element_type=jnp.float32)

    @pl.when(k == _NBLK - 1)
    def _fin():
        g = _elu(acc_ref[...] + b1_ref[...])
        f = _elu(jnp.dot(g, W2_ref[...], preferred_element_type=jnp.float32)
                 + b2_ref[...])
        p = jnp.dot(f, W3_ref[...], preferred_element_type=jnp.float32) + b3_ref[...]
        g_ref[...] = g
        f_ref[...] = f
        p_ref[...] = p


@jax.jit
def _head(flat, W1, b1, W2, b2, W3, b3):
    return pl.pallas_call(
        _head_body,
        grid=(_NBLK,),
        in_specs=[
            pl.BlockSpec((1, _BK), lambda k: (0, k)),
            pl.BlockSpec((_BK, FC), lambda k: (k, 0)),
            pl.BlockSpec((1, FC), lambda k: (0, 0)),
            pl.BlockSpec((FC, OMIC), lambda k: (0, 0)),
            pl.BlockSpec((1, OMIC), lambda k: (0, 0)),
            pl.BlockSpec((OMIC, CLS), lambda k: (0, 0)),
            pl.BlockSpec((1, CLS), lambda k: (0, 0)),
        ],
        out_specs=[
            pl.BlockSpec((1, FC), lambda k: (0, 0)),
            pl.BlockSpec((1, OMIC), lambda k: (0, 0)),
            pl.BlockSpec((1, CLS), lambda k: (0, 0)),
        ],
        out_shape=[
            jax.ShapeDtypeStruct((1, FC), jnp.float32),
            jax.ShapeDtypeStruct((1, OMIC), jnp.float32),
            jax.ShapeDtypeStruct((1, CLS), jnp.float32),
        ],
        scratch_shapes=[pltpu.VMEM((1, FC), jnp.float32)],
    )(flat, W1, b1.reshape(1, FC), W2, b2.reshape(1, OMIC), W3,
      b3.reshape(1, CLS))


def kernel(x, edge_index, W_gat, a_src, a_dst, gamma, beta,
           W1, b1, W2, b2, W3, b3):
    src = edge_index[0]
    dst = edge_index[1]
    rows = jnp.arange(HF)
    cols = rows // F
    As = jnp.zeros((HF, H), jnp.float32).at[rows, cols].set(a_src.reshape(HF))
    Ad = jnp.zeros((HF, H), jnp.float32).at[rows, cols].set(a_dst.reshape(HF))
    stab, dtab, c16 = _tabs(x, W_gat, As, Ad)
    zeros2560 = jnp.zeros((_AR, _TW), jnp.float32)
    acc4 = _edge(src, dst, stab, dtab, c16.reshape(_DW), zeros2560)
    acc = acc4.reshape(_NP * _NC * _AR, _TW)[:N]
    R = jnp.kron(jnp.eye(H, dtype=jnp.float32), jnp.ones((1, F), jnp.float32))
    ln = _post(acc, R, gamma.reshape(1, HF), beta.reshape(1, HF))
    flat = ln.reshape(1, N * HF)
    g, f, p = _head(flat, W1, b1, W2, b2, W3, b3)
    pred = p
    Y_hat = jnp.argmax(pred, axis=1)
    Y_prob = jax.nn.softmax(pred, axis=1)
    return pred, Y_prob, Y_hat, g.reshape(FC), f.reshape(OMIC)


# speedup vs baseline: 50.7136x; 1.1146x over previous
import functools

import jax
import jax.numpy as jnp
from jax import lax
from jax.experimental import pallas as pl
from jax.experimental.pallas import tpu as pltpu
from jax.experimental.pallas import tpu_sc as plsc

N = 10000
E = 320000
D = 128
H = 8
F = 8
HF = H * F
FC = 64
OMIC = 32
CLS = 2

_TW = 128
_DW = 16
_NC = 2
_NS = 16
_NW = _NC * _NS
_ETP = E // _NS
_SB = 80
_NSB = 5
_CH = _SB * _NSB
_NCH = _ETP // _CH
_NP = 2
_AR = 2560
_ZR = _AR // _NS


def _elu(v):
    return jnp.where(v > 0, v, jnp.exp(jnp.minimum(v, 0.0)) - 1.0)


_BN = 1000
_NBN = N // _BN


def _tabs_body(x_ref, wg_ref, as_ref, ad_ref, st_ref, dt_ref, c16_ref,
               ms_ref, md_ref):
    k = pl.program_id(0)
    h = jnp.dot(x_ref[...], wg_ref[...], preferred_element_type=jnp.float32)
    asb = jnp.dot(h, as_ref[...], preferred_element_type=jnp.float32)
    adb = jnp.dot(h, ad_ref[...], preferred_element_type=jnp.float32)
    z56 = jnp.zeros((_BN, _TW - HF - H), jnp.float32)
    st_ref[...] = jnp.concatenate([h, asb, z56], axis=1)
    z120 = jnp.zeros((_BN, _TW - H), jnp.float32)
    dt_ref[...] = jnp.concatenate([adb, z120], axis=1)
    bs = jnp.max(asb, axis=0, keepdims=True)
    bd = jnp.max(adb, axis=0, keepdims=True)

    @pl.when(k == 0)
    def _first():
        ms_ref[...] = bs
        md_ref[...] = bd

    @pl.when(k > 0)
    def _rest():
        ms_ref[...] = jnp.maximum(ms_ref[...], bs)
        md_ref[...] = jnp.maximum(md_ref[...], bd)

    @pl.when(k == _NBN - 1)
    def _fin():
        cv = jnp.maximum(ms_ref[...] + md_ref[...], 0.0)
        c16_ref[...] = jnp.concatenate(
            [cv, jnp.full((1, 8), 1e30, jnp.float32)], axis=1)


@jax.jit
def _tabs(x, wg, As, Ad):
    return pl.pallas_call(
        _tabs_body,
        grid=(_NBN,),
        in_specs=[
            pl.BlockSpec((_BN, D), lambda k: (k, 0)),
            pl.BlockSpec((D, HF), lambda k: (0, 0)),
            pl.BlockSpec((HF, H), lambda k: (0, 0)),
            pl.BlockSpec((HF, H), lambda k: (0, 0)),
        ],
        out_specs=[
            pl.BlockSpec((_BN, _TW), lambda k: (k, 0)),
            pl.BlockSpec((_BN, _TW), lambda k: (k, 0)),
            pl.BlockSpec((1, _DW), lambda k: (0, 0)),
        ],
        out_shape=[
            jax.ShapeDtypeStruct((N, _TW), jnp.float32),
            jax.ShapeDtypeStruct((N, _TW), jnp.float32),
            jax.ShapeDtypeStruct((1, _DW), jnp.float32),
        ],
        scratch_shapes=[pltpu.VMEM((1, H), jnp.float32),
                        pltpu.VMEM((1, H), jnp.float32)],
    )(x, wg, As, Ad)


def _edge_body(src_hbm, dst_hbm, stab_hbm, dtab_hbm, c16_hbm, zeros_hbm,
               out_hbm, sidx, didx, lsidx, lidx, ldidx, gbuf, dbuf, cbuf,
               acc, sem1, sem2, sem3):
    c = lax.axis_index("c")
    s = lax.axis_index("s")

    pltpu.sync_copy(c16_hbm, cbuf)
    cv = cbuf[...]
    io = lax.iota(jnp.int32, 16)
    hi = jnp.where(io >= 8, 1, 0)

    for p in range(_NP):
        rng = 2 * p + c
        nbase = pl.multiple_of(rng * _AR, 8)
        pltpu.sync_copy(zeros_hbm.at[pl.ds(s * _ZR, _ZR)],
                        acc.at[pl.ds(s * _ZR, _ZR)])
        plsc.subcore_barrier()

        def _drain_scatters():
            for q in range(_NSB):
                pltpu.make_async_copy(
                    gbuf.at[pl.ds(q * _SB, _SB)],
                    acc.at[plsc.Indices(lidx.at[q], ignored_value=-1)],
                    sem3).wait()

        def chunk_body(chunk, carry0):
            base = pl.multiple_of(s * _ETP, 8) + chunk * _CH
            pltpu.sync_copy(src_hbm.at[pl.ds(base, _CH)], sidx)
            pltpu.sync_copy(dst_hbm.at[pl.ds(base, _CH)], didx)

            @pl.when(chunk > 0)
            def _d():
                _drain_scatters()

            @plsc.parallel_loop(0, _CH // 16, unroll=5)
            def mask_body(j):
                o16 = pl.multiple_of(j * 16, 16)
                dvv = didx[pl.ds(o16, 16)]
                svv = sidx[pl.ds(o16, 16)]
                lv = dvv - nbase
                ok = jnp.logical_and(lv >= 0, lv < _AR)
                q = j // (_SB // 16)
                r16 = pl.multiple_of((j % (_SB // 16)) * 16, 16)
                lidx[q, pl.ds(r16, 16)] = jnp.where(ok, lv, -1)
                lsidx[q, pl.ds(r16, 16)] = jnp.where(ok, svv, -1)
                ldidx[q, pl.ds(r16, 16)] = jnp.where(ok, dvv, -1)
            cps = []
            for q in range(_NSB):
                cps.append(pltpu.async_copy(
                    stab_hbm.at[plsc.Indices(lsidx.at[q], ignored_value=-1)],
                    gbuf.at[pl.ds(q * _SB, _SB)], sem1))
                cps.append(pltpu.async_copy(
                    dtab_hbm.at[plsc.Indices(ldidx.at[q], ignored_value=-1)],
                    dbuf.at[pl.ds(q * _SB, _SB)], sem2))
            for cp in cps:
                cp.wait()

            @plsc.parallel_loop(0, _CH, unroll=8)
            def body(i):
                dv = dbuf[i, pl.ds(0, 16)]
                av = gbuf[i, pl.ds(HF, 16)]
                sv = av + dv
                lkv = jnp.maximum(sv, 0.2 * sv)
                w = jnp.exp(lkv - cv)
                gbuf[i, pl.ds(HF, 16)] = w
                for u in range(4):
                    hv = gbuf[i, pl.ds(u * 16, 16)]
                    pat = hi + 2 * u
                    wx = jnp.take_along_axis(w, pat, axis=0,
                                             mode="promise_in_bounds")
                    gbuf[i, pl.ds(u * 16, 16)] = hv * wx
            for q in range(_NSB):
                pltpu.async_copy(
                    gbuf.at[pl.ds(q * _SB, _SB)],
                    acc.at[plsc.Indices(lidx.at[q], ignored_value=-1)],
                    sem3, add=True)
            return carry0

        lax.fori_loop(0, _NCH, chunk_body, 0)
        _drain_scatters()

        plsc.subcore_barrier()
        pltpu.sync_copy(acc.at[pl.ds(s * _ZR, _ZR)],
                        out_hbm.at[rng, pl.ds(s * _ZR, _ZR)])
        plsc.subcore_barrier()


@jax.jit
def _edge(src, dst, stab, dtab, c16, zeros2560):
    mesh = plsc.VectorSubcoreMesh(core_axis_name="c", subcore_axis_name="s")
    fn = functools.partial(
        pl.kernel,
        out_type=jax.ShapeDtypeStruct((_NP * _NC, _AR, _TW), jnp.float32),
        mesh=mesh,
        scratch_types=[
            pltpu.VMEM((_CH,), jnp.int32),
            pltpu.VMEM((_CH,), jnp.int32),
            pltpu.VMEM((_NSB, _SB), jnp.int32),
            pltpu.VMEM((_NSB, _SB), jnp.int32),
            pltpu.VMEM((_NSB, _SB), jnp.int32),
            pltpu.VMEM((_CH, _TW), jnp.float32),
            pltpu.VMEM((_CH, _TW), jnp.float32),
            pltpu.VMEM((_DW,), jnp.float32),
            pltpu.VMEM_SHARED((_AR, _TW), jnp.float32),
            pltpu.SemaphoreType.DMA,
            pltpu.SemaphoreType.DMA,
            pltpu.SemaphoreType.DMA,
        ],
    )(_edge_body)
    return fn(src, dst, stab, dtab, c16, zeros2560)


def _post_body(a_ref, r_ref, g_ref, b_ref, o_ref):
    a = a_ref[...]
    numer = a[:, :HF]
    den = a[:, HF:HF + H]
    dexp = jnp.dot(den, r_ref[...], preferred_element_type=jnp.float32)
    o = numer / (dexp + 1e-16)
    o = _elu(o)
    mu = jnp.mean(o, axis=1, keepdims=True)
    var = jnp.mean((o - mu) ** 2, axis=1, keepdims=True)
    o_ref[...] = ((o - mu) * lax.rsqrt(var + 1e-5)) * g_ref[...] + b_ref[...]


@jax.jit
def _post(a, R, gamma, beta):
    return pl.pallas_call(
        _post_body,
        grid=(_NBN,),
        in_specs=[
            pl.BlockSpec((_BN, _TW), lambda k: (k, 0)),
            pl.BlockSpec((H, HF), lambda k: (0, 0)),
            pl.BlockSpec((1, HF), lambda k: (0, 0)),
            pl.BlockSpec((1, HF), lambda k: (0, 0)),
        ],
        out_specs=pl.BlockSpec((_BN, HF), lambda k: (k, 0)),
        out_shape=jax.ShapeDtypeStruct((N, HF), jnp.float32),
    )(a, R, gamma, beta)


_BK = 16000
_NBLK = (N * HF) // _BK


def _head_body(flat_ref, W1_ref, b1_ref, W2_ref, b2_ref, W3_ref, b3_ref,
               g_ref, f_ref, p_ref, acc_ref):
    k = pl.program_id(0)

    @pl.when(k == 0)
    def _init():
        acc_ref[...] = jnp.zeros_like(acc_ref)

    acc_ref[...] += jnp.dot(flat_ref[...], W1_ref[...],
                            preferred_element_type=jnp.float32)

    @pl.when(k == _NBLK - 1)
    def _fin():
        g = _elu(acc_ref[...] + b1_ref[...])
        f = _elu(jnp.dot(g, W2_ref[...], preferred_element_type=jnp.float32)
                 + b2_ref[...])
        p = jnp.dot(f, W3_ref[...], preferred_element_type=jnp.float32) + b3_ref[...]
        g_ref[...] = g
        f_ref[...] = f
        p_ref[...] = p


@jax.jit
def _head(flat, W1, b1, W2, b2, W3, b3):
    return pl.pallas_call(
        _head_body,
        grid=(_NBLK,),
        in_specs=[
            pl.BlockSpec((1, _BK), lambda k: (0, k)),
            pl.BlockSpec((_BK, FC), lambda k: (k, 0)),
            pl.BlockSpec((1, FC), lambda k: (0, 0)),
            pl.BlockSpec((FC, OMIC), lambda k: (0, 0)),
            pl.BlockSpec((1, OMIC), lambda k: (0, 0)),
            pl.BlockSpec((OMIC, CLS), lambda k: (0, 0)),
            pl.BlockSpec((1, CLS), lambda k: (0, 0)),
        ],
        out_specs=[
            pl.BlockSpec((1, FC), lambda k: (0, 0)),
            pl.BlockSpec((1, OMIC), lambda k: (0, 0)),
            pl.BlockSpec((1, CLS), lambda k: (0, 0)),
        ],
        out_shape=[
            jax.ShapeDtypeStruct((1, FC), jnp.float32),
            jax.ShapeDtypeStruct((1, OMIC), jnp.float32),
            jax.ShapeDtypeStruct((1, CLS), jnp.float32),
        ],
        scratch_shapes=[pltpu.VMEM((1, FC), jnp.float32)],
    )(flat, W1, b1.reshape(1, FC), W2, b2.reshape(1, OMIC), W3,
      b3.reshape(1, CLS))


def kernel(x, edge_index, W_gat, a_src, a_dst, gamma, beta,
           W1, b1, W2, b2, W3, b3):
    src = edge_index[0]
    dst = edge_index[1]
    rows = jnp.arange(HF)
    cols = rows // F
    As = jnp.zeros((HF, H), jnp.float32).at[rows, cols].set(a_src.reshape(HF))
    Ad = jnp.zeros((HF, H), jnp.float32).at[rows, cols].set(a_dst.reshape(HF))
    stab, dtab, c16 = _tabs(x, W_gat, As, Ad)
    zeros2560 = jnp.zeros((_AR, _TW), jnp.float32)
    acc4 = _edge(src, dst, stab, dtab, c16.reshape(_DW), zeros2560)
    acc = acc4.reshape(_NP * _NC * _AR, _TW)[:N]
    R = jnp.kron(jnp.eye(H, dtype=jnp.float32), jnp.ones((1, F), jnp.float32))
    ln = _post(acc, R, gamma.reshape(1, HF), beta.reshape(1, HF))
    flat = ln.reshape(1, N * HF)
    g, f, p = _head(flat, W1, b1, W2, b2, W3, b3)
    pred = p
    Y_hat = jnp.argmax(pred, axis=1)
    Y_prob = jax.nn.softmax(pred, axis=1)
    return pred, Y_prob, Y_hat, g.reshape(FC), f.reshape(OMIC)


# speedup vs baseline: 68.9707x; 1.3600x over previous
import functools

import jax
import jax.numpy as jnp
from jax import lax
from jax.experimental import pallas as pl
from jax.experimental.pallas import tpu as pltpu
from jax.experimental.pallas import tpu_sc as plsc

N = 10000
E = 320000
D = 128
H = 8
F = 8
HF = H * F
FC = 64
OMIC = 32
CLS = 2

_TW = 128
_DW = 16
_NC = 2
_NS = 16
_NW = _NC * _NS
_ETP = E // _NS
_SB = 80
_NSB = 5
_CH = _SB * _NSB
_NCH = _ETP // _CH
_NP = 2
_AR = 2560
_ZR = _AR // _NS


def _elu(v):
    return jnp.where(v > 0, v, jnp.exp(jnp.minimum(v, 0.0)) - 1.0)


_BN = 1000
_NBN = N // _BN


def _tabs_body(x_ref, wg_ref, as_ref, ad_ref, st_ref, dt_ref, c16_ref,
               ms_ref, md_ref):
    k = pl.program_id(0)
    h = jnp.dot(x_ref[...], wg_ref[...], preferred_element_type=jnp.float32)
    asb = jnp.dot(h, as_ref[...], preferred_element_type=jnp.float32)
    adb = jnp.dot(h, ad_ref[...], preferred_element_type=jnp.float32)
    z56 = jnp.zeros((_BN, _TW - HF - H), jnp.float32)
    st_ref[...] = jnp.concatenate([h, asb, z56], axis=1)
    z120 = jnp.zeros((_BN, _TW - H), jnp.float32)
    dt_ref[...] = jnp.concatenate([adb, z120], axis=1)
    bs = jnp.max(asb, axis=0, keepdims=True)
    bd = jnp.max(adb, axis=0, keepdims=True)

    @pl.when(k == 0)
    def _first():
        ms_ref[...] = bs
        md_ref[...] = bd

    @pl.when(k > 0)
    def _rest():
        ms_ref[...] = jnp.maximum(ms_ref[...], bs)
        md_ref[...] = jnp.maximum(md_ref[...], bd)

    @pl.when(k == _NBN - 1)
    def _fin():
        cv = jnp.maximum(ms_ref[...] + md_ref[...], 0.0)
        c16_ref[...] = jnp.concatenate(
            [cv, jnp.full((1, 8), 1e30, jnp.float32)], axis=1)


@jax.jit
def _tabs(x, wg, As, Ad):
    return pl.pallas_call(
        _tabs_body,
        grid=(_NBN,),
        in_specs=[
            pl.BlockSpec((_BN, D), lambda k: (k, 0)),
            pl.BlockSpec((D, HF), lambda k: (0, 0)),
            pl.BlockSpec((HF, H), lambda k: (0, 0)),
            pl.BlockSpec((HF, H), lambda k: (0, 0)),
        ],
        out_specs=[
            pl.BlockSpec((_BN, _TW), lambda k: (k, 0)),
            pl.BlockSpec((_BN, _TW), lambda k: (k, 0)),
            pl.BlockSpec((1, _DW), lambda k: (0, 0)),
        ],
        out_shape=[
            jax.ShapeDtypeStruct((N, _TW), jnp.float32),
            jax.ShapeDtypeStruct((N, _TW), jnp.float32),
            jax.ShapeDtypeStruct((1, _DW), jnp.float32),
        ],
        scratch_shapes=[pltpu.VMEM((1, H), jnp.float32),
                        pltpu.VMEM((1, H), jnp.float32)],
    )(x, wg, As, Ad)


def _edge_body(src_hbm, dst_hbm, stab_hbm, dtab_hbm, c16_hbm, zeros_hbm,
               out_hbm, sidx, didx, lsidx, lidx, ldidx, gbuf, dbuf, cbuf,
               acc, sem3, semi, *gsems):
    c = lax.axis_index("c")
    s = lax.axis_index("s")

    pltpu.sync_copy(c16_hbm, cbuf)
    cv = cbuf[...]
    io = lax.iota(jnp.int32, 16)
    hi = jnp.where(io >= 8, 1, 0)

    def _fire_idx(base):
        pltpu.async_copy(src_hbm.at[pl.ds(base, _CH)], sidx, semi)
        pltpu.async_copy(dst_hbm.at[pl.ds(base, _CH)], didx, semi)

    def _wait_idx(base):
        pltpu.make_async_copy(src_hbm.at[pl.ds(base, _CH)], sidx, semi).wait()
        pltpu.make_async_copy(dst_hbm.at[pl.ds(base, _CH)], didx, semi).wait()

    for p in range(_NP):
        rng = 2 * p + c
        nbase = pl.multiple_of(rng * _AR, 8)
        pltpu.sync_copy(zeros_hbm.at[pl.ds(s * _ZR, _ZR)],
                        acc.at[pl.ds(s * _ZR, _ZR)])
        plsc.subcore_barrier()

        ebase = pl.multiple_of(s * _ETP, 8)

        def _drain_scatters():
            for q in range(_NSB):
                pltpu.make_async_copy(
                    gbuf.at[pl.ds(q * _SB, _SB)],
                    acc.at[plsc.Indices(lidx.at[q], ignored_value=-1)],
                    sem3).wait()

        _fire_idx(ebase)

        def chunk_body(chunk, carry0):
            base = ebase + chunk * _CH
            _wait_idx(base)

            @pl.when(chunk > 0)
            def _d():
                _drain_scatters()

            @plsc.parallel_loop(0, _CH // 16, unroll=5)
            def mask_body(j):
                o16 = pl.multiple_of(j * 16, 16)
                dvv = didx[pl.ds(o16, 16)]
                svv = sidx[pl.ds(o16, 16)]
                lv = dvv - nbase
                ok = jnp.logical_and(lv >= 0, lv < _AR)
                q = j // (_SB // 16)
                r16 = pl.multiple_of((j % (_SB // 16)) * 16, 16)
                lidx[q, pl.ds(r16, 16)] = jnp.where(ok, lv, -1)
                lsidx[q, pl.ds(r16, 16)] = jnp.where(ok, svv, -1)
                ldidx[q, pl.ds(r16, 16)] = jnp.where(ok, dvv, -1)

            for q in range(_NSB):
                pltpu.async_copy(
                    stab_hbm.at[plsc.Indices(lsidx.at[q], ignored_value=-1)],
                    gbuf.at[pl.ds(q * _SB, _SB)], gsems[2 * q])
                pltpu.async_copy(
                    dtab_hbm.at[plsc.Indices(ldidx.at[q], ignored_value=-1)],
                    dbuf.at[pl.ds(q * _SB, _SB)], gsems[2 * q + 1])

            @pl.when(chunk + 1 < _NCH)
            def _pf():
                _fire_idx(base + _CH)

            for q in range(_NSB):
                pltpu.make_async_copy(
                    stab_hbm.at[plsc.Indices(lsidx.at[q], ignored_value=-1)],
                    gbuf.at[pl.ds(q * _SB, _SB)], gsems[2 * q]).wait()
                pltpu.make_async_copy(
                    dtab_hbm.at[plsc.Indices(ldidx.at[q], ignored_value=-1)],
                    dbuf.at[pl.ds(q * _SB, _SB)], gsems[2 * q + 1]).wait()

                @plsc.parallel_loop(q * _SB, (q + 1) * _SB, unroll=8)
                def body(i):
                    dv = dbuf[i, pl.ds(0, 16)]
                    av = gbuf[i, pl.ds(HF, 16)]
                    sv = av + dv
                    lkv = jnp.maximum(sv, 0.2 * sv)
                    w = jnp.exp(lkv - cv)
                    gbuf[i, pl.ds(HF, 16)] = w
                    for u in range(4):
                        hv = gbuf[i, pl.ds(u * 16, 16)]
                        pat = hi + 2 * u
                        wx = jnp.take_along_axis(w, pat, axis=0,
                                                 mode="promise_in_bounds")
                        gbuf[i, pl.ds(u * 16, 16)] = hv * wx

                pltpu.async_copy(
                    gbuf.at[pl.ds(q * _SB, _SB)],
                    acc.at[plsc.Indices(lidx.at[q], ignored_value=-1)],
                    sem3, add=True)
            return carry0

        lax.fori_loop(0, _NCH, chunk_body, 0)
        _drain_scatters()

        plsc.subcore_barrier()
        pltpu.sync_copy(acc.at[pl.ds(s * _ZR, _ZR)],
                        out_hbm.at[rng, pl.ds(s * _ZR, _ZR)])
        plsc.subcore_barrier()


@jax.jit
def _edge(src, dst, stab, dtab, c16, zeros2560):
    mesh = plsc.VectorSubcoreMesh(core_axis_name="c", subcore_axis_name="s")
    fn = functools.partial(
        pl.kernel,
        out_type=jax.ShapeDtypeStruct((_NP * _NC, _AR, _TW), jnp.float32),
        mesh=mesh,
        scratch_types=[
            pltpu.VMEM((_CH,), jnp.int32),
            pltpu.VMEM((_CH,), jnp.int32),
            pltpu.VMEM((_NSB, _SB), jnp.int32),
            pltpu.VMEM((_NSB, _SB), jnp.int32),
            pltpu.VMEM((_NSB, _SB), jnp.int32),
            pltpu.VMEM((_CH, _TW), jnp.float32),
            pltpu.VMEM((_CH, _TW), jnp.float32),
            pltpu.VMEM((_DW,), jnp.float32),
            pltpu.VMEM_SHARED((_AR, _TW), jnp.float32),
        ] + [pltpu.SemaphoreType.DMA] * (2 + 2 * _NSB),
    )(_edge_body)
    return fn(src, dst, stab, dtab, c16, zeros2560)


def _post_body(a_ref, r_ref, g_ref, b_ref, o_ref):
    a = a_ref[...]
    numer = a[:, :HF]
    den = a[:, HF:HF + H]
    dexp = jnp.dot(den, r_ref[...], preferred_element_type=jnp.float32)
    o = numer / (dexp + 1e-16)
    o = _elu(o)
    mu = jnp.mean(o, axis=1, keepdims=True)
    var = jnp.mean((o - mu) ** 2, axis=1, keepdims=True)
    o_ref[...] = ((o - mu) * lax.rsqrt(var + 1e-5)) * g_ref[...] + b_ref[...]


@jax.jit
def _post(a, R, gamma, beta):
    return pl.pallas_call(
        _post_body,
        grid=(_NBN,),
        in_specs=[
            pl.BlockSpec((_BN, _TW), lambda k: (k, 0)),
            pl.BlockSpec((H, HF), lambda k: (0, 0)),
            pl.BlockSpec((1, HF), lambda k: (0, 0)),
            pl.BlockSpec((1, HF), lambda k: (0, 0)),
        ],
        out_specs=pl.BlockSpec((_BN, HF), lambda k: (k, 0)),
        out_shape=jax.ShapeDtypeStruct((N, HF), jnp.float32),
    )(a, R, gamma, beta)


_BK = 16000
_NBLK = (N * HF) // _BK


def _head_body(flat_ref, W1_ref, b1_ref, W2_ref, b2_ref, W3_ref, b3_ref,
               g_ref, f_ref, p_ref, acc_ref):
    k = pl.program_id(0)

    @pl.when(k == 0)
    def _init():
        acc_ref[...] = jnp.zeros_like(acc_ref)

    acc_ref[...] += jnp.dot(flat_ref[...], W1_ref[...],
                            preferred_element_type=jnp.float32)

    @pl.when(k == _NBLK - 1)
    def _fin():
        g = _elu(acc_ref[...] + b1_ref[...])
        f = _elu(jnp.dot(g, W2_ref[...], preferred_element_type=jnp.float32)
                 + b2_ref[...])
        p = jnp.dot(f, W3_ref[...], preferred_element_type=jnp.float32) + b3_ref[...]
        g_ref[...] = g
        f_ref[...] = f
        p_ref[...] = p


@jax.jit
def _head(flat, W1, b1, W2, b2, W3, b3):
    return pl.pallas_call(
        _head_body,
        grid=(_NBLK,),
        in_specs=[
            pl.BlockSpec((1, _BK), lambda k: (0, k)),
            pl.BlockSpec((_BK, FC), lambda k: (k, 0)),
            pl.BlockSpec((1, FC), lambda k: (0, 0)),
            pl.BlockSpec((FC, OMIC), lambda k: (0, 0)),
            pl.BlockSpec((1, OMIC), lambda k: (0, 0)),
            pl.BlockSpec((OMIC, CLS), lambda k: (0, 0)),
            pl.BlockSpec((1, CLS), lambda k: (0, 0)),
        ],
        out_specs=[
            pl.BlockSpec((1, FC), lambda k: (0, 0)),
            pl.BlockSpec((1, OMIC), lambda k: (0, 0)),
            pl.BlockSpec((1, CLS), lambda k: (0, 0)),
        ],
        out_shape=[
            jax.ShapeDtypeStruct((1, FC), jnp.float32),
            jax.ShapeDtypeStruct((1, OMIC), jnp.float32),
            jax.ShapeDtypeStruct((1, CLS), jnp.float32),
        ],
        scratch_shapes=[pltpu.VMEM((1, FC), jnp.float32)],
    )(flat, W1, b1.reshape(1, FC), W2, b2.reshape(1, OMIC), W3,
      b3.reshape(1, CLS))


def kernel(x, edge_index, W_gat, a_src, a_dst, gamma, beta,
           W1, b1, W2, b2, W3, b3):
    src = edge_index[0]
    dst = edge_index[1]
    rows = jnp.arange(HF)
    cols = rows // F
    As = jnp.zeros((HF, H), jnp.float32).at[rows, cols].set(a_src.reshape(HF))
    Ad = jnp.zeros((HF, H), jnp.float32).at[rows, cols].set(a_dst.reshape(HF))
    stab, dtab, c16 = _tabs(x, W_gat, As, Ad)
    zeros2560 = jnp.zeros((_AR, _TW), jnp.float32)
    acc4 = _edge(src, dst, stab, dtab, c16.reshape(_DW), zeros2560)
    acc = acc4.reshape(_NP * _NC * _AR, _TW)[:N]
    R = jnp.kron(jnp.eye(H, dtype=jnp.float32), jnp.ones((1, F), jnp.float32))
    ln = _post(acc, R, gamma.reshape(1, HF), beta.reshape(1, HF))
    flat = ln.reshape(1, N * HF)
    g, f, p = _head(flat, W1, b1, W2, b2, W3, b3)
    pred = p
    Y_hat = jnp.argmax(pred, axis=1)
    Y_prob = jax.nn.softmax(pred, axis=1)
    return pred, Y_prob, Y_hat, g.reshape(FC), f.reshape(OMIC)
